# baseline (speedup 1.0000x reference)
import functools

import jax
import jax.numpy as jnp
from jax import lax
from jax.experimental import pallas as pl
from jax.experimental.pallas import tpu as pltpu
from jax.experimental.pallas import tpu_sc as plsc

N = 10000
E = 320000
D = 128
BN_EPS = 1e-5

NC = 2
NS = 16
NW = NC * NS
CH = 128
T_CH = 80
H_CH = T_CH // 2
M_CH = E // CH
M_CHP = (NW - 1) * T_CH + H_CH
NROWS = 10240
RPT = NROWS // NS

_mesh = plsc.VectorSubcoreMesh(
    core_axis_name="c", subcore_axis_name="s", num_cores=NC, num_subcores=NS)


def _wid():
    return lax.axis_index("s") * NC + lax.axis_index("c")


def _deg_body(col2d, ones_hbm, zeros_hbm, degp, idx_v, ones_v, sem, deg_sh):
    c = lax.axis_index("c")
    s = lax.axis_index("s")
    w = _wid()
    last = NW - 1
    pltpu.sync_copy(zeros_hbm, deg_sh.at[pl.ds(s * RPT, RPT)])
    pltpu.sync_copy(ones_hbm, ones_v)
    plsc.subcore_barrier()

    for h in range(2):
        base = w * T_CH + h * H_CH
        if h == 0:
            pltpu.sync_copy(col2d.at[pl.ds(base, H_CH)], idx_v)
            nj = H_CH
        else:
            @pl.when(w < last)
            def _():
                pltpu.sync_copy(col2d.at[pl.ds(base, H_CH)], idx_v)

            nj = jnp.where(w == last, 0, H_CH)

        def step(j, carry):
            pltpu.async_copy(ones_v, deg_sh.at[idx_v.at[j]], sem, add=True)
            return carry

        lax.fori_loop(0, nj, step, 0)

        def drain(j, carry):
            pltpu.make_async_copy(ones_v, deg_sh.at[idx_v.at[0]], sem).wait()
            return carry

        lax.fori_loop(0, nj, drain, 0)
    plsc.subcore_barrier()
    pltpu.sync_copy(deg_sh.at[pl.ds(s * RPT, RPT)],
                    degp.at[pl.ds(c * NROWS + s * RPT, RPT)])


_deg_kernel = functools.partial(
    pl.kernel,
    out_type=jax.ShapeDtypeStruct((NC * NROWS,), jnp.float32),
    mesh=_mesh,
    scratch_types=[
        pltpu.VMEM((H_CH, CH), jnp.int32),
        pltpu.VMEM((CH,), jnp.float32),
        pltpu.SemaphoreType.DMA,
        pltpu.VMEM_SHARED((NROWS,), jnp.float32),
    ],
)(_deg_body)


def _seg_body(hs_hbm, row2d, col2d, zeros_hbm, accp,
              ridx_v, cidx_v, rows_a, rows_b, sem_a, sem_b, acc_sh):
    c = lax.axis_index("c")
    s = lax.axis_index("s")
    w = _wid()
    pltpu.sync_copy(zeros_hbm, acc_sh.at[pl.ds(s * RPT, RPT)])
    plsc.subcore_barrier()
    last = NW - 1

    for h in range(T_CH // H_CH):
        base = w * T_CH + h * H_CH

        if h == 0:
            pltpu.sync_copy(row2d.at[pl.ds(base, H_CH)], ridx_v)
            pltpu.sync_copy(col2d.at[pl.ds(base, H_CH)], cidx_v)
            npair = H_CH // 2
        else:
            @pl.when(w < last)
            def _():
                pltpu.sync_copy(row2d.at[pl.ds(base, H_CH)], ridx_v)
                pltpu.sync_copy(col2d.at[pl.ds(base, H_CH)], cidx_v)

            npair = jnp.where(w == last, 0, H_CH // 2)

        @pl.when(npair > 0)
        def _():
            pltpu.async_copy(hs_hbm.at[ridx_v.at[0]], rows_a, sem_a)

        def step(jj, carry):
            j = 2 * jj
            pltpu.async_copy(hs_hbm.at[ridx_v.at[j + 1]], rows_b, sem_b)
            pltpu.make_async_copy(hs_hbm.at[ridx_v.at[j]], rows_a, sem_a).wait()
            pltpu.sync_copy(rows_a, acc_sh.at[cidx_v.at[j]], add=True)

            @pl.when(jj < npair - 1)
            def _():
                pltpu.async_copy(hs_hbm.at[ridx_v.at[j + 2]], rows_a, sem_a)

            pltpu.make_async_copy(hs_hbm.at[ridx_v.at[j + 1]], rows_b,
                                  sem_b).wait()
            pltpu.sync_copy(rows_b, acc_sh.at[cidx_v.at[j + 1]], add=True)
            return carry

        lax.fori_loop(0, npair, step, 0)
    plsc.subcore_barrier()
    pltpu.sync_copy(acc_sh.at[pl.ds(s * RPT, RPT)],
                    accp.at[c, pl.ds(s * RPT, RPT)])


_seg_kernel = functools.partial(
    pl.kernel,
    out_type=jax.ShapeDtypeStruct((NC, NROWS, D), jnp.float32),
    mesh=_mesh,
    scratch_types=[
        pltpu.VMEM((H_CH, CH), jnp.int32),
        pltpu.VMEM((H_CH, CH), jnp.int32),
        pltpu.VMEM((CH, D), jnp.float32),
        pltpu.VMEM((CH, D), jnp.float32),
        pltpu.SemaphoreType.DMA,
        pltpu.SemaphoreType.DMA,
        pltpu.VMEM_SHARED((NROWS, D), jnp.float32),
    ],
)(_seg_body)


BR = 1000


def _mm_body(x_ref, w_ref, degp_ref, hs_ref):
    h = jnp.dot(x_ref[...], w_ref[...], preferred_element_type=jnp.float32)
    d = degp_ref[0] + degp_ref[1]
    dis = lax.rsqrt(d + 1.0)
    hs_ref[...] = h * dis


def _d1_body(accp_ref, hs_ref, degp_ref, b_ref, outp_ref, stats_ref, s1, s2):
    i = pl.program_id(0)
    acc = accp_ref[0] + accp_ref[1]
    d = degp_ref[0] + degp_ref[1]
    dis = lax.rsqrt(d + 1.0)
    o = dis * (acc + hs_ref[...]) + b_ref[...]
    outp_ref[...] = o

    @pl.when(i == 0)
    def _():
        s1[...] = jnp.zeros_like(s1)
        s2[...] = jnp.zeros_like(s2)

    s1[...] += jnp.sum(o, axis=0, keepdims=True)
    s2[...] += jnp.sum(o * o, axis=0, keepdims=True)

    @pl.when(i == pl.num_programs(0) - 1)
    def _():
        stats_ref[0:1] = s1[...]
        stats_ref[1:2] = s2[...]


def _d2_body(outp_ref, stats_ref, bnw_ref, bnb_ref, out_ref):
    n = jnp.float32(N)
    mean = stats_ref[0:1] / n
    var = stats_ref[1:2] / n - mean * mean
    inv = lax.rsqrt(var + BN_EPS)
    out_ref[...] = ((outp_ref[...] - mean) * inv) * bnw_ref[...] + bnb_ref[...]


def kernel(x, edge_index, W, b, bn_weight, bn_bias):
    row = edge_index[0].astype(jnp.int32)
    col = edge_index[1].astype(jnp.int32)
    npadc = M_CHP - M_CH
    dummy_c = N + (jnp.arange(npadc * CH, dtype=jnp.int32) % (NROWS - N))
    row2d = jnp.concatenate(
        [row.reshape(M_CH, CH), jnp.zeros((npadc, CH), jnp.int32)])
    col2d = jnp.concatenate(
        [col.reshape(M_CH, CH), dummy_c.reshape(npadc, CH)])

    ones_hbm = jnp.ones((CH,), jnp.float32)
    zeros_deg = jnp.zeros((RPT,), jnp.float32)
    zeros_acc = jnp.zeros((RPT, D), jnp.float32)

    degp = _deg_kernel(col2d, ones_hbm, zeros_deg).reshape(NC, NROWS, 1)

    hs = pl.pallas_call(
        _mm_body,
        grid=(N // BR,),
        in_specs=[
            pl.BlockSpec((BR, D), lambda i: (i, 0)),
            pl.BlockSpec((D, D), lambda i: (0, 0)),
            pl.BlockSpec((NC, BR, 1), lambda i: (0, i, 0)),
        ],
        out_specs=pl.BlockSpec((BR, D), lambda i: (i, 0)),
        out_shape=jax.ShapeDtypeStruct((N, D), jnp.float32),
    )(x, W, degp)

    accp = _seg_kernel(hs, row2d, col2d, zeros_acc)

    outp, stats = pl.pallas_call(
        _d1_body,
        grid=(N // BR,),
        in_specs=[
            pl.BlockSpec((NC, BR, D), lambda i: (0, i, 0)),
            pl.BlockSpec((BR, D), lambda i: (i, 0)),
            pl.BlockSpec((NC, BR, 1), lambda i: (0, i, 0)),
            pl.BlockSpec((1, D), lambda i: (0, 0)),
        ],
        out_specs=[
            pl.BlockSpec((BR, D), lambda i: (i, 0)),
            pl.BlockSpec((2, D), lambda i: (0, 0)),
        ],
        out_shape=[
            jax.ShapeDtypeStruct((N, D), jnp.float32),
            jax.ShapeDtypeStruct((2, D), jnp.float32),
        ],
        scratch_shapes=[
            pltpu.VMEM((1, D), jnp.float32),
            pltpu.VMEM((1, D), jnp.float32),
        ],
    )(accp, hs, degp, b.reshape(1, D))

    out = pl.pallas_call(
        _d2_body,
        grid=(N // BR,),
        in_specs=[
            pl.BlockSpec((BR, D), lambda i: (i, 0)),
            pl.BlockSpec((2, D), lambda i: (0, 0)),
            pl.BlockSpec((1, D), lambda i: (0, 0)),
            pl.BlockSpec((1, D), lambda i: (0, 0)),
        ],
        out_specs=pl.BlockSpec((BR, D), lambda i: (i, 0)),
        out_shape=jax.ShapeDtypeStruct((N, D), jnp.float32),
    )(outp, stats, bn_weight.reshape(1, D), bn_bias.reshape(1, D))
    return out


# speedup vs baseline: 1.5749x; 1.5749x over previous
import functools

import jax
import jax.numpy as jnp
from jax import lax
from jax.experimental import pallas as pl
from jax.experimental.pallas import tpu as pltpu
from jax.experimental.pallas import tpu_sc as plsc

N = 10000
E = 320000
D = 128
BN_EPS = 1e-5

NC = 2
NS = 16
NW = NC * NS
CH = 128
T_CH = 80
H_CH = T_CH // 2
M_CH = E // CH
M_CHP = (NW - 1) * T_CH + H_CH
NROWS = 10240
RPT = NROWS // NS

_mesh = plsc.VectorSubcoreMesh(
    core_axis_name="c", subcore_axis_name="s", num_cores=NC, num_subcores=NS)


def _wid():
    return lax.axis_index("s") * NC + lax.axis_index("c")


def _deg_body(col2d, ones_hbm, zeros_hbm, degp, idx_v, ones_v, sem, deg_sh):
    c = lax.axis_index("c")
    s = lax.axis_index("s")
    w = _wid()
    last = NW - 1
    pltpu.sync_copy(zeros_hbm, deg_sh.at[pl.ds(s * RPT, RPT)])
    pltpu.sync_copy(ones_hbm, ones_v)
    plsc.subcore_barrier()

    for h in range(2):
        base = w * T_CH + h * H_CH
        if h == 0:
            pltpu.sync_copy(col2d.at[pl.ds(base, H_CH)], idx_v)
            nj = H_CH
        else:
            @pl.when(w < last)
            def _():
                pltpu.sync_copy(col2d.at[pl.ds(base, H_CH)], idx_v)

            nj = jnp.where(w == last, 0, H_CH)

        def step(j, carry):
            pltpu.async_copy(ones_v, deg_sh.at[idx_v.at[j]], sem, add=True)
            return carry

        lax.fori_loop(0, nj, step, 0)

        def drain(j, carry):
            pltpu.make_async_copy(ones_v, deg_sh.at[idx_v.at[0]], sem).wait()
            return carry

        lax.fori_loop(0, nj, drain, 0)
    plsc.subcore_barrier()
    pltpu.sync_copy(deg_sh.at[pl.ds(s * RPT, RPT)],
                    degp.at[pl.ds(c * NROWS + s * RPT, RPT)])


_deg_kernel = functools.partial(
    pl.kernel,
    out_type=jax.ShapeDtypeStruct((NC * NROWS,), jnp.float32),
    mesh=_mesh,
    scratch_types=[
        pltpu.VMEM((H_CH, CH), jnp.int32),
        pltpu.VMEM((CH,), jnp.float32),
        pltpu.SemaphoreType.DMA,
        pltpu.VMEM_SHARED((NROWS,), jnp.float32),
    ],
)(_deg_body)


def _seg_body(hs_hbm, row2d, col2d, zeros_hbm, accp,
              ridx_v, cidx_v, rows_a, rows_b, sem_a, sem_b, acc_sh):
    c = lax.axis_index("c")
    s = lax.axis_index("s")
    w = _wid()
    pltpu.sync_copy(zeros_hbm, acc_sh.at[pl.ds(s * RPT, RPT)])
    plsc.subcore_barrier()
    last = NW - 1

    for h in range(T_CH // H_CH):
        base = w * T_CH + h * H_CH

        if h == 0:
            pltpu.sync_copy(row2d.at[pl.ds(base, H_CH)], ridx_v)
            pltpu.sync_copy(col2d.at[pl.ds(base, H_CH)], cidx_v)
            npair = H_CH // 2
        else:
            @pl.when(w < last)
            def _():
                pltpu.sync_copy(row2d.at[pl.ds(base, H_CH)], ridx_v)
                pltpu.sync_copy(col2d.at[pl.ds(base, H_CH)], cidx_v)

            npair = jnp.where(w == last, 0, H_CH // 2)

        @pl.when(npair > 0)
        def _():
            pltpu.async_copy(hs_hbm.at[ridx_v.at[0]], rows_a, sem_a)

        def step(jj, carry):
            j = 2 * jj
            pltpu.async_copy(hs_hbm.at[ridx_v.at[j + 1]], rows_b, sem_b)
            pltpu.make_async_copy(hs_hbm.at[ridx_v.at[j]], rows_a, sem_a).wait()
            pltpu.sync_copy(rows_a, acc_sh.at[cidx_v.at[j]], add=True)

            @pl.when(jj < npair - 1)
            def _():
                pltpu.async_copy(hs_hbm.at[ridx_v.at[j + 2]], rows_a, sem_a)

            pltpu.make_async_copy(hs_hbm.at[ridx_v.at[j + 1]], rows_b,
                                  sem_b).wait()
            pltpu.sync_copy(rows_b, acc_sh.at[cidx_v.at[j + 1]], add=True)
            return carry

        lax.fori_loop(0, npair, step, 0)
    plsc.subcore_barrier()
    pltpu.sync_copy(acc_sh.at[pl.ds(s * RPT, RPT)],
                    accp.at[c, pl.ds(s * RPT, RPT)])


_seg_kernel = functools.partial(
    pl.kernel,
    out_type=jax.ShapeDtypeStruct((NC, NROWS, D), jnp.float32),
    mesh=_mesh,
    scratch_types=[
        pltpu.VMEM((H_CH, CH), jnp.int32),
        pltpu.VMEM((H_CH, CH), jnp.int32),
        pltpu.VMEM((CH, D), jnp.float32),
        pltpu.VMEM((CH, D), jnp.float32),
        pltpu.SemaphoreType.DMA,
        pltpu.SemaphoreType.DMA,
        pltpu.VMEM_SHARED((NROWS, D), jnp.float32),
    ],
)(_seg_body)


BR = 1000


def _mm_body(x_ref, w_ref, degp_ref, hs_ref):
    h = jnp.dot(x_ref[...], w_ref[...], preferred_element_type=jnp.float32)
    d = degp_ref[0] + degp_ref[1]
    dis = lax.rsqrt(d + 1.0)
    hs_ref[...] = h * dis


def _d1_body(accp_ref, hs_ref, degp_ref, b_ref, outp_ref, stats_ref, s1, s2):
    i = pl.program_id(0)
    acc = accp_ref[0] + accp_ref[1]
    d = degp_ref[0] + degp_ref[1]
    dis = lax.rsqrt(d + 1.0)
    o = dis * (acc + hs_ref[...]) + b_ref[...]
    outp_ref[...] = o

    @pl.when(i == 0)
    def _():
        s1[...] = jnp.zeros_like(s1)
        s2[...] = jnp.zeros_like(s2)

    s1[...] += jnp.sum(o, axis=0, keepdims=True)
    s2[...] += jnp.sum(o * o, axis=0, keepdims=True)

    @pl.when(i == pl.num_programs(0) - 1)
    def _():
        stats_ref[0:1] = s1[...]
        stats_ref[1:2] = s2[...]


def _d2_body(outp_ref, stats_ref, bnw_ref, bnb_ref, out_ref):
    n = jnp.float32(N)
    mean = stats_ref[0:1] / n
    var = stats_ref[1:2] / n - mean * mean
    inv = lax.rsqrt(var + BN_EPS)
    out_ref[...] = ((outp_ref[...] - mean) * inv) * bnw_ref[...] + bnb_ref[...]


def kernel(x, edge_index, W, b, bn_weight, bn_bias):
    row = edge_index[0].astype(jnp.int32)
    col = edge_index[1].astype(jnp.int32)
    npadc = M_CHP - M_CH
    dummy_r = jnp.arange(npadc * CH, dtype=jnp.int32) * 3 % N
    dummy_c = N + (jnp.arange(npadc * CH, dtype=jnp.int32) % (NROWS - N))
    row2d = jnp.concatenate(
        [row.reshape(M_CH, CH), dummy_r.reshape(npadc, CH)])
    col2d = jnp.concatenate(
        [col.reshape(M_CH, CH), dummy_c.reshape(npadc, CH)])

    ones_hbm = jnp.ones((CH,), jnp.float32)
    zeros_deg = jnp.zeros((RPT,), jnp.float32)
    zeros_acc = jnp.zeros((RPT, D), jnp.float32)

    degp = _deg_kernel(col2d, ones_hbm, zeros_deg).reshape(NC, NROWS, 1)

    hs = pl.pallas_call(
        _mm_body,
        grid=(N // BR,),
        in_specs=[
            pl.BlockSpec((BR, D), lambda i: (i, 0)),
            pl.BlockSpec((D, D), lambda i: (0, 0)),
            pl.BlockSpec((NC, BR, 1), lambda i: (0, i, 0)),
        ],
        out_specs=pl.BlockSpec((BR, D), lambda i: (i, 0)),
        out_shape=jax.ShapeDtypeStruct((N, D), jnp.float32),
    )(x, W, degp)

    accp = _seg_kernel(hs, row2d, col2d, zeros_acc)

    outp, stats = pl.pallas_call(
        _d1_body,
        grid=(N // BR,),
        in_specs=[
            pl.BlockSpec((NC, BR, D), lambda i: (0, i, 0)),
            pl.BlockSpec((BR, D), lambda i: (i, 0)),
            pl.BlockSpec((NC, BR, 1), lambda i: (0, i, 0)),
            pl.BlockSpec((1, D), lambda i: (0, 0)),
        ],
        out_specs=[
            pl.BlockSpec((BR, D), lambda i: (i, 0)),
            pl.BlockSpec((2, D), lambda i: (0, 0)),
        ],
        out_shape=[
            jax.ShapeDtypeStruct((N, D), jnp.float32),
            jax.ShapeDtypeStruct((2, D), jnp.float32),
        ],
        scratch_shapes=[
            pltpu.VMEM((1, D), jnp.float32),
            pltpu.VMEM((1, D), jnp.float32),
        ],
    )(accp, hs, degp, b.reshape(1, D))

    out = pl.pallas_call(
        _d2_body,
        grid=(N // BR,),
        in_specs=[
            pl.BlockSpec((BR, D), lambda i: (i, 0)),
            pl.BlockSpec((2, D), lambda i: (0, 0)),
            pl.BlockSpec((1, D), lambda i: (0, 0)),
            pl.BlockSpec((1, D), lambda i: (0, 0)),
        ],
        out_specs=pl.BlockSpec((BR, D), lambda i: (i, 0)),
        out_shape=jax.ShapeDtypeStruct((N, D), jnp.float32),
    )(outp, stats, bn_weight.reshape(1, D), bn_bias.reshape(1, D))
    return out


# speedup vs baseline: 1.5811x; 1.0040x over previous
import functools

import jax
import jax.numpy as jnp
from jax import lax
from jax.experimental import pallas as pl
from jax.experimental.pallas import tpu as pltpu
from jax.experimental.pallas import tpu_sc as plsc

N = 10000
E = 320000
D = 128
BN_EPS = 1e-5

NC = 2
NS = 16
NW = NC * NS
CH = 128
T_CH = 80
H_CH = T_CH // 2
M_CH = E // CH
M_CHP = (NW - 1) * T_CH + H_CH
NROWS = 10240
RPT = NROWS // NS

_mesh = plsc.VectorSubcoreMesh(
    core_axis_name="c", subcore_axis_name="s", num_cores=NC, num_subcores=NS)


def _wid():
    return lax.axis_index("s") * NC + lax.axis_index("c")


def _deg_body(col2d, tail_c, ones_hbm, zeros_hbm, degp, idx_v, ones_v, sem, deg_sh):
    c = lax.axis_index("c")
    s = lax.axis_index("s")
    w = _wid()
    last = NW - 1
    pltpu.sync_copy(zeros_hbm, deg_sh.at[pl.ds(s * RPT, RPT)])
    pltpu.sync_copy(ones_hbm, ones_v)
    plsc.subcore_barrier()

    for h in range(2):
        base = w * T_CH + h * H_CH
        if h == 0:
            @pl.when(w < last)
            def _():
                pltpu.sync_copy(col2d.at[pl.ds(base, H_CH)], idx_v)

            @pl.when(w == last)
            def _():
                pltpu.sync_copy(tail_c, idx_v)

            nj = H_CH
        else:
            @pl.when(w < last)
            def _():
                pltpu.sync_copy(col2d.at[pl.ds(base, H_CH)], idx_v)

            nj = jnp.where(w == last, 0, H_CH)

        def step(j, carry):
            pltpu.async_copy(ones_v, deg_sh.at[idx_v.at[j]], sem, add=True)
            return carry

        lax.fori_loop(0, nj, step, 0)

        def drain(j, carry):
            pltpu.make_async_copy(ones_v, deg_sh.at[idx_v.at[0]], sem).wait()
            return carry

        lax.fori_loop(0, nj, drain, 0)
    plsc.subcore_barrier()
    pltpu.sync_copy(deg_sh.at[pl.ds(s * RPT, RPT)],
                    degp.at[pl.ds(c * NROWS + s * RPT, RPT)])


_deg_kernel = functools.partial(
    pl.kernel,
    out_type=jax.ShapeDtypeStruct((NC * NROWS,), jnp.float32),
    mesh=_mesh,
    scratch_types=[
        pltpu.VMEM((H_CH, CH), jnp.int32),
        pltpu.VMEM((CH,), jnp.float32),
        pltpu.SemaphoreType.DMA,
        pltpu.VMEM_SHARED((NROWS,), jnp.float32),
    ],
)(_deg_body)


def _seg_body(hs_hbm, row2d, col2d, tail_r, tail_c, zeros_hbm, accp,
              ridx_v, cidx_v, rows_a, rows_b, sem_a, sem_b, acc_sh):
    c = lax.axis_index("c")
    s = lax.axis_index("s")
    w = _wid()
    pltpu.sync_copy(zeros_hbm, acc_sh.at[pl.ds(s * RPT, RPT)])
    plsc.subcore_barrier()
    last = NW - 1

    for h in range(T_CH // H_CH):
        base = w * T_CH + h * H_CH

        if h == 0:
            @pl.when(w < last)
            def _():
                pltpu.sync_copy(row2d.at[pl.ds(base, H_CH)], ridx_v)
                pltpu.sync_copy(col2d.at[pl.ds(base, H_CH)], cidx_v)

            @pl.when(w == last)
            def _():
                pltpu.sync_copy(tail_r, ridx_v)
                pltpu.sync_copy(tail_c, cidx_v)

            npair = H_CH // 2
        else:
            @pl.when(w < last)
            def _():
                pltpu.sync_copy(row2d.at[pl.ds(base, H_CH)], ridx_v)
                pltpu.sync_copy(col2d.at[pl.ds(base, H_CH)], cidx_v)

            npair = jnp.where(w == last, 0, H_CH // 2)

        @pl.when(npair > 0)
        def _():
            pltpu.async_copy(hs_hbm.at[ridx_v.at[0]], rows_a, sem_a)

        def step(jj, carry):
            j = 2 * jj
            pltpu.async_copy(hs_hbm.at[ridx_v.at[j + 1]], rows_b, sem_b)
            pltpu.make_async_copy(hs_hbm.at[ridx_v.at[j]], rows_a, sem_a).wait()
            pltpu.sync_copy(rows_a, acc_sh.at[cidx_v.at[j]], add=True)

            @pl.when(jj < npair - 1)
            def _():
                pltpu.async_copy(hs_hbm.at[ridx_v.at[j + 2]], rows_a, sem_a)

            pltpu.make_async_copy(hs_hbm.at[ridx_v.at[j + 1]], rows_b,
                                  sem_b).wait()
            pltpu.sync_copy(rows_b, acc_sh.at[cidx_v.at[j + 1]], add=True)
            return carry

        lax.fori_loop(0, npair, step, 0)
    plsc.subcore_barrier()
    pltpu.sync_copy(acc_sh.at[pl.ds(s * RPT, RPT)],
                    accp.at[c, pl.ds(s * RPT, RPT)])


_seg_kernel = functools.partial(
    pl.kernel,
    out_type=jax.ShapeDtypeStruct((NC, NROWS, D), jnp.float32),
    mesh=_mesh,
    scratch_types=[
        pltpu.VMEM((H_CH, CH), jnp.int32),
        pltpu.VMEM((H_CH, CH), jnp.int32),
        pltpu.VMEM((CH, D), jnp.float32),
        pltpu.VMEM((CH, D), jnp.float32),
        pltpu.SemaphoreType.DMA,
        pltpu.SemaphoreType.DMA,
        pltpu.VMEM_SHARED((NROWS, D), jnp.float32),
    ],
)(_seg_body)


BR = 1000


def _mm_body(x_ref, w_ref, degp_ref, hs_ref):
    h = jnp.dot(x_ref[...], w_ref[...], preferred_element_type=jnp.float32)
    d = degp_ref[0] + degp_ref[1]
    dis = lax.rsqrt(d + 1.0)
    hs_ref[...] = h * dis


def _d1_body(accp_ref, hs_ref, degp_ref, b_ref, outp_ref, stats_ref, s1, s2):
    i = pl.program_id(0)
    acc = accp_ref[0] + accp_ref[1]
    d = degp_ref[0] + degp_ref[1]
    dis = lax.rsqrt(d + 1.0)
    o = dis * (acc + hs_ref[...]) + b_ref[...]
    outp_ref[...] = o

    @pl.when(i == 0)
    def _():
        s1[...] = jnp.zeros_like(s1)
        s2[...] = jnp.zeros_like(s2)

    s1[...] += jnp.sum(o, axis=0, keepdims=True)
    s2[...] += jnp.sum(o * o, axis=0, keepdims=True)

    @pl.when(i == pl.num_programs(0) - 1)
    def _():
        stats_ref[0:1] = s1[...]
        stats_ref[1:2] = s2[...]


def _d2_body(outp_ref, stats_ref, bnw_ref, bnb_ref, out_ref):
    n = jnp.float32(N)
    mean = stats_ref[0:1] / n
    var = stats_ref[1:2] / n - mean * mean
    inv = lax.rsqrt(var + BN_EPS)
    out_ref[...] = ((outp_ref[...] - mean) * inv) * bnw_ref[...] + bnb_ref[...]


def kernel(x, edge_index, W, b, bn_weight, bn_bias):
    row = edge_index[0].astype(jnp.int32)
    col = edge_index[1].astype(jnp.int32)
    row2d = row.reshape(M_CH, CH)
    col2d = col.reshape(M_CH, CH)
    npadc = H_CH - (M_CH - (NW - 1) * T_CH)
    dummy_r = jnp.arange(npadc * CH, dtype=jnp.int32) * 3 % N
    dummy_c = N + (jnp.arange(npadc * CH, dtype=jnp.int32) % (NROWS - N))
    tail_r = jnp.concatenate(
        [row2d[(NW - 1) * T_CH:], dummy_r.reshape(npadc, CH)])
    tail_c = jnp.concatenate(
        [col2d[(NW - 1) * T_CH:], dummy_c.reshape(npadc, CH)])

    ones_hbm = jnp.ones((CH,), jnp.float32)
    zeros_deg = jnp.zeros((RPT,), jnp.float32)
    zeros_acc = jnp.zeros((RPT, D), jnp.float32)

    degp = _deg_kernel(col2d, tail_c, ones_hbm, zeros_deg).reshape(NC, NROWS, 1)

    hs = pl.pallas_call(
        _mm_body,
        grid=(N // BR,),
        in_specs=[
            pl.BlockSpec((BR, D), lambda i: (i, 0)),
            pl.BlockSpec((D, D), lambda i: (0, 0)),
            pl.BlockSpec((NC, BR, 1), lambda i: (0, i, 0)),
        ],
        out_specs=pl.BlockSpec((BR, D), lambda i: (i, 0)),
        out_shape=jax.ShapeDtypeStruct((N, D), jnp.float32),
    )(x, W, degp)

    accp = _seg_kernel(hs, row2d, col2d, tail_r, tail_c, zeros_acc)

    outp, stats = pl.pallas_call(
        _d1_body,
        grid=(N // BR,),
        in_specs=[
            pl.BlockSpec((NC, BR, D), lambda i: (0, i, 0)),
            pl.BlockSpec((BR, D), lambda i: (i, 0)),
            pl.BlockSpec((NC, BR, 1), lambda i: (0, i, 0)),
            pl.BlockSpec((1, D), lambda i: (0, 0)),
        ],
        out_specs=[
            pl.BlockSpec((BR, D), lambda i: (i, 0)),
            pl.BlockSpec((2, D), lambda i: (0, 0)),
        ],
        out_shape=[
            jax.ShapeDtypeStruct((N, D), jnp.float32),
            jax.ShapeDtypeStruct((2, D), jnp.float32),
        ],
        scratch_shapes=[
            pltpu.VMEM((1, D), jnp.float32),
            pltpu.VMEM((1, D), jnp.float32),
        ],
    )(accp, hs, degp, b.reshape(1, D))

    out = pl.pallas_call(
        _d2_body,
        grid=(N // BR,),
        in_specs=[
            pl.BlockSpec((BR, D), lambda i: (i, 0)),
            pl.BlockSpec((2, D), lambda i: (0, 0)),
            pl.BlockSpec((1, D), lambda i: (0, 0)),
            pl.BlockSpec((1, D), lambda i: (0, 0)),
        ],
        out_specs=pl.BlockSpec((BR, D), lambda i: (i, 0)),
        out_shape=jax.ShapeDtypeStruct((N, D), jnp.float32),
    )(outp, stats, bn_weight.reshape(1, D), bn_bias.reshape(1, D))
    return out
